# baseline (speedup 1.0000x reference)
import jax
import jax.numpy as jnp
from jax.experimental import pallas as pl
from jax.experimental.pallas import tpu as pltpu

ACTIVE_KERNELS = 4
NCHUNK = 16


def _dot_t(a, b):
    return jax.lax.dot_general(a, b, (((1,), (1,)), ((), ())),
                               preferred_element_type=jnp.float32)


def _dot_s(a, b):
    return jax.lax.dot_general(a, b, (((1,), (0,)), ((), ())),
                               preferred_element_type=jnp.float32)


def _rim_body(x_ref, h_ref, wq_ref, wk_ref, wv_ref, wih_ref, whh_ref, out_ref,
              kqh_s, kql_s, m_s, d_s, wx_s):
    c = pl.program_id(1)
    h = h_ref[0]

    @pl.when(c == 0)
    def _init():
        wq = wq_ref[...]
        wk = wk_ref[...]
        q = jnp.sum(h[:, :, None] * wq, axis=1)
        kq = jnp.sum(wk * q[:, None, :], axis=2)
        kqh = kq.astype(jnp.bfloat16)
        kqh_s[...] = kqh
        kql_s[...] = (kq - kqh.astype(jnp.float32)).astype(jnp.bfloat16)
        m_s[...] = jnp.zeros_like(m_s)
        d_s[...] = jnp.ones_like(d_s)
        wx_s[...] = jnp.zeros_like(wx_s)

    x = x_ref[0]
    xh = x.astype(jnp.bfloat16)
    xl = (x - xh.astype(jnp.float32)).astype(jnp.bfloat16)
    kqh = kqh_s[...]
    kql = kql_s[...]
    sim = _dot_t(kqh, xh) + (_dot_t(kqh, xl) + _dot_t(kql, xh))

    m_old = m_s[...]
    mc = jnp.max(sim, axis=1, keepdims=True)
    m_new = jnp.maximum(m_old, mc)
    alpha = jnp.exp(m_old - m_new)
    p = jnp.exp(sim - m_new)
    pb = p.astype(jnp.bfloat16)
    m_s[...] = m_new
    d_s[...] = d_s[...] * alpha + jnp.sum(p, axis=1, keepdims=True)
    wx_s[...] = wx_s[...] * alpha + (_dot_s(pb, xh) + _dot_s(pb, xl))

    @pl.when(c == NCHUNK - 1)
    def _finish():
        wx = wx_s[...] / d_s[...]
        wv = wv_ref[...]
        wih = wih_ref[...]
        whh = whh_ref[...]
        attended = jnp.sum(wx[:, :, None] * wv, axis=1)
        pre = jnp.sum(attended[:, :, None] * wih, axis=1) \
            + jnp.sum(h[:, :, None] * whh, axis=1)
        new_h = jnp.tanh(pre)
        k_idx = jax.lax.broadcasted_iota(jnp.int32, new_h.shape, 0)
        out_ref[0] = jnp.where(k_idx < ACTIVE_KERNELS, new_h, h)


def kernel(input, rim_hidden_states, hidden_to_query_map, input_to_key_map,
           input_to_values_map, w_ih, w_hh, interpret=False):
    B, S, D = input.shape
    K, H = rim_hidden_states.shape[1], rim_hidden_states.shape[2]
    A = hidden_to_query_map.shape[2]
    C = S // NCHUNK

    return pl.pallas_call(
        _rim_body,
        grid=(B, NCHUNK),
        in_specs=[
            pl.BlockSpec((1, C, D), lambda b, c: (b, c, 0)),
            pl.BlockSpec((1, K, H), lambda b, c: (b, 0, 0)),
            pl.BlockSpec((K, H, A), lambda b, c: (0, 0, 0)),
            pl.BlockSpec((K, D, A), lambda b, c: (0, 0, 0)),
            pl.BlockSpec((K, D, A), lambda b, c: (0, 0, 0)),
            pl.BlockSpec((K, A, H), lambda b, c: (0, 0, 0)),
            pl.BlockSpec((K, H, H), lambda b, c: (0, 0, 0)),
        ],
        out_specs=pl.BlockSpec((1, K, H), lambda b, c: (b, 0, 0)),
        out_shape=jax.ShapeDtypeStruct((B, K, H), jnp.float32),
        scratch_shapes=[
            pltpu.VMEM((K, D), jnp.bfloat16),
            pltpu.VMEM((K, D), jnp.bfloat16),
            pltpu.VMEM((K, 1), jnp.float32),
            pltpu.VMEM((K, 1), jnp.float32),
            pltpu.VMEM((K, D), jnp.float32),
        ],
        interpret=interpret,
    )(input, rim_hidden_states, hidden_to_query_map, input_to_key_map,
      input_to_values_map, w_ih, w_hh)


# speedup vs baseline: 1.7407x; 1.7407x over previous
import jax
import jax.numpy as jnp
from jax.experimental import pallas as pl
from jax.experimental.pallas import tpu as pltpu

ACTIVE_KERNELS = 4
NCB = 4
_B, _S, _D, _K = 4, 8192, 64, 16
_C = _S // NCB
_NTOT = _B * NCB


def _dot_t(a, b):
    return jax.lax.dot_general(a, b, (((1,), (1,)), ((), ())),
                               preferred_element_type=jnp.float32)


def _dot_s(a, b):
    return jax.lax.dot_general(a, b, (((1,), (0,)), ((), ())),
                               preferred_element_type=jnp.float32)


def _rim_body(x_hbm, h_ref, wq_ref, wk_ref, wv_ref, wih_ref, whh_ref, out_ref,
              xbuf, sem, kqh_s, kql_s):
    wq = wq_ref[...]
    wk = wk_ref[...]

    def copy(i, slot):
        b = i // NCB
        c = jax.lax.rem(i, NCB)
        return pltpu.make_async_copy(
            x_hbm.at[b, pl.ds(c * _C, _C), :], xbuf.at[slot], sem.at[slot])

    copy(0, 0).start()

    for b in range(_B):
        hb = h_ref[b]
        q = jnp.sum(hb[:, :, None] * wq, axis=1)
        kq = jnp.sum(wk * q[:, None, :], axis=2)
        kqh = kq.astype(jnp.bfloat16)
        kqh_s[b * _K:(b + 1) * _K, :] = kqh
        kql_s[b * _K:(b + 1) * _K, :] = (kq - kqh.astype(jnp.float32)).astype(jnp.bfloat16)

    def chunk(i, carry):
        m_old, d_old, wx_old = carry
        slot = jax.lax.rem(i, 2)
        b = i // NCB
        c = jax.lax.rem(i, NCB)

        @pl.when(i + 1 < _NTOT)
        def _():
            copy(i + 1, 1 - slot).start()

        copy(i, slot).wait()
        x = xbuf[slot]
        xh = x.astype(jnp.bfloat16)
        xl = (x - xh.astype(jnp.float32)).astype(jnp.bfloat16)
        kqh = kqh_s[pl.ds(b * _K, _K), :]
        kql = kql_s[pl.ds(b * _K, _K), :]
        sim = _dot_t(kqh, xh) + (_dot_t(kqh, xl) + _dot_t(kql, xh))

        first = (c == 0)
        m_old = jnp.where(first, jnp.zeros_like(m_old), m_old)
        d_old = jnp.where(first, jnp.ones_like(d_old), d_old)
        wx_old = jnp.where(first, jnp.zeros_like(wx_old), wx_old)

        mc = jnp.max(sim, axis=1, keepdims=True)
        m_new = jnp.maximum(m_old, mc)
        alpha = jnp.exp(m_old - m_new)
        p = jnp.exp(sim - m_new)
        pb = p.astype(jnp.bfloat16)
        d_new = d_old * alpha + jnp.sum(p, axis=1, keepdims=True)
        wx_new = wx_old * alpha + (_dot_s(pb, xh) + _dot_s(pb, xl))

        @pl.when(c == NCB - 1)
        def _():
            hb = h_ref[b]
            wx = wx_new / d_new
            attended = jnp.sum(wx[:, :, None] * wv_ref[...], axis=1)
            pre = jnp.sum(attended[:, :, None] * wih_ref[...], axis=1) \
                + jnp.sum(hb[:, :, None] * whh_ref[...], axis=1)
            new_h = jnp.tanh(pre)
            k_idx = jax.lax.broadcasted_iota(jnp.int32, new_h.shape, 0)
            out_ref[pl.ds(b, 1), :, :] = jnp.where(
                k_idx < ACTIVE_KERNELS, new_h, hb)[None]

        return m_new, d_new, wx_new

    init = (jnp.zeros((_K, 1), jnp.float32),
            jnp.ones((_K, 1), jnp.float32),
            jnp.zeros((_K, _D), jnp.float32))
    jax.lax.fori_loop(0, _NTOT, chunk, init)


def kernel(input, rim_hidden_states, hidden_to_query_map, input_to_key_map,
           input_to_values_map, w_ih, w_hh, interpret=False):
    B, S, D = input.shape
    K, H = rim_hidden_states.shape[1], rim_hidden_states.shape[2]

    return pl.pallas_call(
        _rim_body,
        in_specs=[
            pl.BlockSpec(memory_space=pl.ANY),
            pl.BlockSpec(memory_space=pltpu.MemorySpace.VMEM),
            pl.BlockSpec(memory_space=pltpu.MemorySpace.VMEM),
            pl.BlockSpec(memory_space=pltpu.MemorySpace.VMEM),
            pl.BlockSpec(memory_space=pltpu.MemorySpace.VMEM),
            pl.BlockSpec(memory_space=pltpu.MemorySpace.VMEM),
            pl.BlockSpec(memory_space=pltpu.MemorySpace.VMEM),
        ],
        out_specs=pl.BlockSpec(memory_space=pltpu.MemorySpace.VMEM),
        out_shape=jax.ShapeDtypeStruct((B, K, H), jnp.float32),
        scratch_shapes=[
            pltpu.VMEM((2, _C, _D), jnp.float32),
            pltpu.SemaphoreType.DMA((2,)),
            pltpu.VMEM((_B * _K, _D), jnp.bfloat16),
            pltpu.VMEM((_B * _K, _D), jnp.bfloat16),
        ],
        interpret=interpret,
    )(input, rim_hidden_states, hidden_to_query_map, input_to_key_map,
      input_to_values_map, w_ih, w_hh)


# speedup vs baseline: 2.0720x; 1.1904x over previous
import jax
import jax.numpy as jnp
from jax.experimental import pallas as pl

ACTIVE_KERNELS = 4


def _dot_t(a, b):
    return jax.lax.dot_general(a, b, (((1,), (1,)), ((), ())),
                               preferred_element_type=jnp.float32)


def _dot_s(a, b):
    return jax.lax.dot_general(a, b, (((1,), (0,)), ((), ())),
                               preferred_element_type=jnp.float32)


def _rim_body(x_ref, h_ref, wq_ref, wk_ref, wv_ref, wih_ref, whh_ref, out_ref):
    x = x_ref[0]
    h = h_ref[0]

    q = jnp.sum(h[:, :, None] * wq_ref[...], axis=1)
    kq = jnp.sum(wk_ref[...] * q[:, None, :], axis=2)

    xh = x.astype(jnp.bfloat16)
    xl = (x - xh.astype(jnp.float32)).astype(jnp.bfloat16)
    kqh = kq.astype(jnp.bfloat16)
    kql = (kq - kqh.astype(jnp.float32)).astype(jnp.bfloat16)

    sim = _dot_t(kqh, xh) + (_dot_t(kqh, xl) + _dot_t(kql, xh))
    m = jnp.maximum(jnp.max(sim, axis=1, keepdims=True), 0.0)
    pb = jnp.exp(sim - m).astype(jnp.bfloat16)
    denom = jnp.sum(pb.astype(jnp.float32), axis=1, keepdims=True) \
        + jnp.exp(-m)
    wx = _dot_s(pb, xh) / denom
    attended = jnp.sum(wx[:, :, None] * wv_ref[...], axis=1)
    pre = jnp.sum(attended[:, :, None] * wih_ref[...], axis=1) \
        + jnp.sum(h[:, :, None] * whh_ref[...], axis=1)
    new_h = jnp.tanh(pre)
    k_idx = jax.lax.broadcasted_iota(jnp.int32, new_h.shape, 0)
    out_ref[0] = jnp.where(k_idx < ACTIVE_KERNELS, new_h, h)


def kernel(input, rim_hidden_states, hidden_to_query_map, input_to_key_map,
           input_to_values_map, w_ih, w_hh, interpret=False):
    B, S, D = input.shape
    K, H = rim_hidden_states.shape[1], rim_hidden_states.shape[2]
    A = hidden_to_query_map.shape[2]

    return pl.pallas_call(
        _rim_body,
        grid=(B,),
        in_specs=[
            pl.BlockSpec((1, S, D), lambda b: (b, 0, 0)),
            pl.BlockSpec((1, K, H), lambda b: (b, 0, 0)),
            pl.BlockSpec((K, H, A), lambda b: (0, 0, 0)),
            pl.BlockSpec((K, D, A), lambda b: (0, 0, 0)),
            pl.BlockSpec((K, D, A), lambda b: (0, 0, 0)),
            pl.BlockSpec((K, A, H), lambda b: (0, 0, 0)),
            pl.BlockSpec((K, H, H), lambda b: (0, 0, 0)),
        ],
        out_specs=pl.BlockSpec((1, K, H), lambda b: (b, 0, 0)),
        out_shape=jax.ShapeDtypeStruct((B, K, H), jnp.float32),
        interpret=interpret,
    )(input, rim_hidden_states, hidden_to_query_map, input_to_key_map,
      input_to_values_map, w_ih, w_hh)
